# baseline (speedup 1.0000x reference)
import functools

import jax
import jax.numpy as jnp
from jax import lax
from jax.experimental import pallas as pl
from jax.experimental.pallas import tpu as pltpu
from jax.experimental.pallas import tpu_sc as plsc

_ETA = 0.8
_GAMMA = 0.9

_NC = 2
_NS = 16
_NW = _NC * _NS
_L = 16
_CHUNK = 8192
_NBUF = 2


@functools.lru_cache(maxsize=None)
def _build(n):
    per_w = n // _NW
    nchunks = per_w // _CHUNK
    assert per_w % _CHUNK == 0
    ng = n // 128

    mesh = plsc.VectorSubcoreMesh(
        core_axis_name="c", subcore_axis_name="s",
        num_cores=_NC, num_subcores=_NS)

    scratch = []
    for _ in range(_NBUF):
        scratch += [
            pltpu.VMEM((64, 2, 128), jnp.float32),
            pltpu.VMEM((64, 2, 128), jnp.float32),
            pltpu.VMEM((16, 4, 128), jnp.int32),
            pltpu.VMEM((16, 4, 128), jnp.int32),
            pltpu.VMEM((16, 4, 128), jnp.float32),
            pltpu.SemaphoreType.DMA,
            pltpu.SemaphoreType.DMA,
        ]

    @functools.partial(
        pl.kernel,
        out_type=jax.ShapeDtypeStruct((2, ng, 2, 128), jnp.float32),
        mesh=mesh,
        scratch_types=scratch,
        compiler_params=pltpu.CompilerParams(needs_layout_passes=False),
    )
    def run(q_hbm, a_hbm, b_hbm, p_hbm, out_hbm, *bufs):
        wid = lax.axis_index("s") * _NC + lax.axis_index("c")
        slots = [bufs[i * 7:(i + 1) * 7] for i in range(_NBUF)]

        def chunk_coords(g):
            cid = wid * nchunks + g
            band = cid // 2
            h = cid % 2
            u0 = cid * (_CHUNK // 128)
            return band, h, u0

        def start_in_q(g, slot):
            q0b, q1b = slot[0], slot[1]
            insem = slot[5]
            _, _, u0 = chunk_coords(g)
            return [
                pltpu.async_copy(q_hbm.at[0, pl.ds(u0, 64)], q0b, insem),
                pltpu.async_copy(q_hbm.at[1, pl.ds(u0, 64)], q1b, insem),
            ]

        def start_in_abp(g, slot):
            ab, bb, pb = slot[2], slot[3], slot[4]
            insem = slot[5]
            band, h, _ = chunk_coords(g)
            return [
                pltpu.async_copy(a_hbm.at[band, :, pl.ds(4 * h, 4), :], ab, insem),
                pltpu.async_copy(b_hbm.at[band, :, pl.ds(4 * h, 4), :], bb, insem),
                pltpu.async_copy(p_hbm.at[band, :, pl.ds(4 * h, 4), :], pb, insem),
            ]

        def start_out(g, slot):
            q0b, q1b = slot[0], slot[1]
            outsem = slot[6]
            _, _, u0 = chunk_coords(g)
            return [
                pltpu.async_copy(q0b, out_hbm.at[0, pl.ds(u0, 64)], outsem),
                pltpu.async_copy(q1b, out_hbm.at[1, pl.ds(u0, 64)], outsem),
            ]

        def compute(slot):
            q0b, q1b, ab, bb, pb, _, _ = slot
            lane = lax.iota(jnp.int32, _L)

            @plsc.parallel_loop(0, _CHUNK // _L, unroll=8)
            def body(j):
                cc = j // 32
                sp = (j // 8) % 4
                l0 = (j % 8) * _L
                u = sp * 16 + cc
                a = ab[cc, sp, pl.ds(l0, _L)]
                b = bb[cc, sp, pl.ds(l0, _L)]
                p = pb[cc, sp, pl.ds(l0, _L)]
                q00 = q0b[u, 0, pl.ds(l0, _L)]
                q01 = q0b[u, 1, pl.ds(l0, _L)]
                q10 = q1b[u, 0, pl.ds(l0, _L)]
                q11 = q1b[u, 1, pl.ds(l0, _L)]
                b0 = b == 0
                a0 = a == 0
                maxv = jnp.where(b0, jnp.maximum(q00, q01),
                                 jnp.maximum(q10, q11))
                qsel = jnp.where(a0, jnp.where(b0, q00, q01),
                                 jnp.where(b0, q10, q11))
                upd = (1.0 - _ETA) * qsel + _ETA * (p + _GAMMA * maxv)
                uv = jnp.broadcast_to(u, (_L,))
                lv = lane + l0
                plsc.store_scatter(q0b, [uv, b, lv], upd, mask=a0)
                plsc.store_scatter(q1b, [uv, b, lv], upd, mask=~a0)

        in_descs = [None] * _NBUF
        out_descs = [None] * _NBUF
        in_descs[0] = start_in_q(0, slots[0]) + start_in_abp(0, slots[0])
        for g in range(nchunks):
            s = g % _NBUF
            ns = (g + 1) % _NBUF
            if g + 1 < nchunks:
                in_descs[ns] = start_in_abp(g + 1, slots[ns])
            for d in in_descs[s]:
                d.wait()
            compute(slots[s])
            if g + 1 < nchunks:
                if out_descs[ns] is not None:
                    for d in out_descs[ns]:
                        d.wait()
                    out_descs[ns] = None
                in_descs[ns] += start_in_q(g + 1, slots[ns])
            out_descs[s] = start_out(g, slots[s])
        for ods in out_descs:
            if ods is not None:
                for d in ods:
                    d.wait()

    return run


def kernel(type_t_matrix, type_t1_matrix, Q_tensor, profit_matrix):
    n = Q_tensor.shape[0]
    ng = n // 128
    qv = Q_tensor.reshape(ng, 128, 2, 2).transpose(2, 0, 3, 1)
    av = type_t_matrix.reshape(256, 8, 16, 128).transpose(0, 2, 1, 3)
    bv = type_t1_matrix.reshape(256, 8, 16, 128).transpose(0, 2, 1, 3)
    pv = profit_matrix.reshape(256, 8, 16, 128).transpose(0, 2, 1, 3)
    out = _build(n)(qv, av.astype(jnp.int32), bv.astype(jnp.int32), pv)
    return out.transpose(1, 3, 0, 2).reshape(n, 2, 2)


# speedup vs baseline: 1.2551x; 1.2551x over previous
import functools

import jax
import jax.numpy as jnp
from jax import lax
from jax.experimental import pallas as pl
from jax.experimental.pallas import tpu as pltpu
from jax.experimental.pallas import tpu_sc as plsc

_ETA = 0.8
_GAMMA = 0.9

_NC = 2
_NS = 16
_NW = _NC * _NS
_L = 16
_CHUNK = 8192
_NBUF = 2


@functools.lru_cache(maxsize=None)
def _build(n):
    per_w = n // _NW
    nchunks = per_w // _CHUNK
    assert per_w % _CHUNK == 0
    ng = n // 128

    mesh = plsc.VectorSubcoreMesh(
        core_axis_name="c", subcore_axis_name="s",
        num_cores=_NC, num_subcores=_NS)

    scratch = []
    for _ in range(_NBUF):
        scratch += [
            pltpu.VMEM((64, 2, 128), jnp.float32),
            pltpu.VMEM((64, 2, 128), jnp.float32),
            pltpu.VMEM((16, 4, 128), jnp.int32),
            pltpu.VMEM((16, 4, 128), jnp.int32),
            pltpu.VMEM((16, 4, 128), jnp.float32),
            pltpu.SemaphoreType.DMA,
            pltpu.SemaphoreType.DMA,
        ]

    @functools.partial(
        pl.kernel,
        out_type=jax.ShapeDtypeStruct((2, ng, 2, 128), jnp.float32),
        mesh=mesh,
        scratch_types=scratch,
        compiler_params=pltpu.CompilerParams(needs_layout_passes=False),
    )
    def run(q_hbm, a_hbm, b_hbm, p_hbm, out_hbm, *bufs):
        wid = lax.axis_index("s") * _NC + lax.axis_index("c")
        slots = [bufs[i * 7:(i + 1) * 7] for i in range(_NBUF)]

        def chunk_coords(g):
            cid = wid * nchunks + g
            band = cid // 2
            h = cid % 2
            u0 = cid * (_CHUNK // 128)
            return band, h, u0

        def start_in(g, slot):
            q0b, q1b, ab, bb, pb, insem, _ = slot
            band, h, u0 = chunk_coords(g)
            return [
                pltpu.async_copy(q_hbm.at[0, pl.ds(u0, 64)], q0b, insem),
                pltpu.async_copy(q_hbm.at[1, pl.ds(u0, 64)], q1b, insem),
                pltpu.async_copy(a_hbm.at[band, :, pl.ds(4 * h, 4), :], ab, insem),
                pltpu.async_copy(b_hbm.at[band, :, pl.ds(4 * h, 4), :], bb, insem),
                pltpu.async_copy(p_hbm.at[band, :, pl.ds(4 * h, 4), :], pb, insem),
            ]

        def start_out(g, slot):
            q0b, q1b = slot[0], slot[1]
            outsem = slot[6]
            _, _, u0 = chunk_coords(g)
            return [
                pltpu.async_copy(q0b, out_hbm.at[0, pl.ds(u0, 64)], outsem),
                pltpu.async_copy(q1b, out_hbm.at[1, pl.ds(u0, 64)], outsem),
            ]

        def compute(slot):
            q0b, q1b, ab, bb, pb, _, _ = slot
            lane = lax.iota(jnp.int32, _L)

            @plsc.parallel_loop(0, _CHUNK // _L, unroll=8)
            def body(j):
                cc = j // 32
                sp = (j // 8) % 4
                l0 = (j % 8) * _L
                u = sp * 16 + cc
                a = ab[cc, sp, pl.ds(l0, _L)]
                b = bb[cc, sp, pl.ds(l0, _L)]
                p = pb[cc, sp, pl.ds(l0, _L)]
                q00 = q0b[u, 0, pl.ds(l0, _L)]
                q01 = q0b[u, 1, pl.ds(l0, _L)]
                q10 = q1b[u, 0, pl.ds(l0, _L)]
                q11 = q1b[u, 1, pl.ds(l0, _L)]
                b0 = b == 0
                a0 = a == 0
                maxv = jnp.where(b0, jnp.maximum(q00, q01),
                                 jnp.maximum(q10, q11))
                qsel = jnp.where(a0, jnp.where(b0, q00, q01),
                                 jnp.where(b0, q10, q11))
                upd = (1.0 - _ETA) * qsel + _ETA * (p + _GAMMA * maxv)
                uv = jnp.broadcast_to(u, (_L,))
                lv = lane + l0
                plsc.store_scatter(q0b, [uv, b, lv], upd, mask=a0)
                plsc.store_scatter(q1b, [uv, b, lv], upd, mask=~a0)

        in_descs = [None] * _NBUF
        out_descs = [None] * _NBUF
        in_descs[0] = start_in(0, slots[0])
        for g in range(nchunks):
            s = g % _NBUF
            ns = (g + 1) % _NBUF
            if g + 1 < nchunks:
                if out_descs[ns] is not None:
                    for d in out_descs[ns]:
                        d.wait()
                    out_descs[ns] = None
                in_descs[ns] = start_in(g + 1, slots[ns])
            for d in in_descs[s]:
                d.wait()
            compute(slots[s])
            out_descs[s] = start_out(g, slots[s])
        for ods in out_descs:
            if ods is not None:
                for d in ods:
                    d.wait()

    return run


def kernel(type_t_matrix, type_t1_matrix, Q_tensor, profit_matrix):
    n = Q_tensor.shape[0]
    ng = n // 128
    qv = Q_tensor.reshape(ng, 128, 2, 2).transpose(2, 0, 3, 1)
    av = type_t_matrix.reshape(256, 8, 16, 128).transpose(0, 2, 1, 3)
    bv = type_t1_matrix.reshape(256, 8, 16, 128).transpose(0, 2, 1, 3)
    pv = profit_matrix.reshape(256, 8, 16, 128).transpose(0, 2, 1, 3)
    out = _build(n)(qv, av.astype(jnp.int32), bv.astype(jnp.int32), pv)
    return out.transpose(1, 3, 0, 2).reshape(n, 2, 2)
